# baseline (speedup 1.0000x reference)
import functools

import jax
import jax.numpy as jnp
from jax import lax
from jax.experimental import pallas as pl
from jax.experimental.pallas import tpu as pltpu
from jax.experimental.pallas import tpu_sc as plsc

BATCH = 16384
EMB_D = 32
NC = 2
NS = 16
LANES = 16
NW = NC * NS
B_PER_W = BATCH // NW
CHUNK = 128
NCHUNK = B_PER_W // CHUNK
GROUPS = B_PER_W // LANES


def _sc_body(uid_hbm, iid_hbm, uemb_hbm, iemb_hbm, ubias_hbm, ibias_hbm,
             gb_hbm, out_hbm, uidx, iidx, urows, irows, ub, ib, gbv, outv,
             sem):
    c = lax.axis_index("c")
    s = lax.axis_index("s")
    wid = s * NC + c

    pltpu.sync_copy(uid_hbm.at[pl.ds(wid * NCHUNK, NCHUNK)], uidx)
    pltpu.sync_copy(iid_hbm.at[pl.ds(wid * NCHUNK, NCHUNK)], iidx)
    pltpu.sync_copy(gb_hbm, gbv)

    handles = []
    for j in range(NCHUNK):
        dst = pl.ds(j * CHUNK, CHUNK)
        handles.append(pltpu.async_copy(uemb_hbm.at[uidx.at[j]], urows.at[dst], sem))
        handles.append(pltpu.async_copy(iemb_hbm.at[iidx.at[j]], irows.at[dst], sem))
        handles.append(pltpu.async_copy(ubias_hbm.at[uidx.at[j]], ub.at[dst], sem))
        handles.append(pltpu.async_copy(ibias_hbm.at[iidx.at[j]], ib.at[dst], sem))
    for h in handles:
        h.wait()

    lane = lax.iota(jnp.int32, LANES)
    gb = gbv[...]

    def group_body(g, carry):
        row = lane + g * LANES
        acc = ub[pl.ds(g * LANES, LANES)] + ib[pl.ds(g * LANES, LANES)] + gb
        for d in range(EMB_D):
            col = jnp.full((LANES,), d, jnp.int32)
            u = plsc.load_gather(urows, [row, col])
            v = plsc.load_gather(irows, [row, col])
            acc = acc + u * v
        outv[pl.ds(g * LANES, LANES)] = acc
        return carry

    lax.fori_loop(0, GROUPS, group_body, 0)

    pltpu.sync_copy(outv, out_hbm.at[pl.ds(wid * B_PER_W, B_PER_W)])


@functools.partial(jax.jit, donate_argnums=())
def _run(uid, iid, uemb, iemb, ubias, ibias, gb16):
    mesh = plsc.VectorSubcoreMesh(core_axis_name="c", subcore_axis_name="s")
    return pl.kernel(
        _sc_body,
        out_type=jax.ShapeDtypeStruct((BATCH,), jnp.float32),
        mesh=mesh,
        scratch_types=[
            pltpu.VMEM((NCHUNK, CHUNK), jnp.int32),
            pltpu.VMEM((NCHUNK, CHUNK), jnp.int32),
            pltpu.VMEM((B_PER_W, EMB_D), jnp.float32),
            pltpu.VMEM((B_PER_W, EMB_D), jnp.float32),
            pltpu.VMEM((B_PER_W,), jnp.float32),
            pltpu.VMEM((B_PER_W,), jnp.float32),
            pltpu.VMEM((LANES,), jnp.float32),
            pltpu.VMEM((B_PER_W,), jnp.float32),
            pltpu.SemaphoreType.DMA,
        ],
        compiler_params=pltpu.CompilerParams(
            needs_layout_passes=False, use_tc_tiling_on_sc=False),
    )(uid, iid, uemb, iemb, ubias, ibias, gb16)


def kernel(user_id, item_id, tag_embedding, user_emb_W, item_emb_W,
           user_bias_W, item_bias_W, global_bias):
    del tag_embedding
    uid = user_id.astype(jnp.int32).reshape(BATCH // CHUNK, CHUNK)
    iid = item_id.astype(jnp.int32).reshape(BATCH // CHUNK, CHUNK)
    ubias = user_bias_W.reshape(-1)
    ibias = item_bias_W.reshape(-1)
    gb16 = jnp.broadcast_to(global_bias.astype(jnp.float32), (LANES,))
    return _run(uid, iid, user_emb_W, item_emb_W, ubias, ibias, gb16)
